# Initial kernel scaffold; baseline (speedup 1.0000x reference)
import functools

import jax
import jax.numpy as jnp
from jax import lax
from jax.experimental import pallas as pl
from jax.experimental.pallas import tpu as pltpu
from jax.experimental.pallas import tpu_sc as plsc

N = 10000
E = 320000
F = 128
H = 64
G = 128

NC = 2
NS = 16
L = 16
NW = NC * NS

EPT = E // NW
CH = 128
NFULL = EPT // CH
TAIL = EPT - NFULL * CH
RPT = N // NS

NPT = 312
REST = N - NW * NPT

_mesh = plsc.VectorSubcoreMesh(core_axis_name="c", subcore_axis_name="s")


@functools.partial(
    pl.kernel,
    out_type=jax.ShapeDtypeStruct((NC, N), jnp.float32),
    mesh=_mesh,
    scratch_types=[
        pltpu.VMEM((CH,), jnp.int32),
        pltpu.VMEM((TAIL,), jnp.int32),
        pltpu.VMEM((CH,), jnp.float32),
        pltpu.VMEM((640,), jnp.float32),
        pltpu.VMEM_SHARED((N,), jnp.float32),
    ],
)
def _deg(dst_hbm, out_hbm, didx, didx_t, ones_v, zb, acc):
    c = lax.axis_index("c")
    s = lax.axis_index("s")
    w = c * NS + s
    base = w * EPT

    def z16(i, carry):
        zb[pl.ds(i * L, L)] = jnp.zeros((L,), jnp.float32)
        return carry

    lax.fori_loop(0, 640 // L, z16, 0)
    for j in range(CH // L):
        ones_v[pl.ds(j * L, L)] = jnp.ones((L,), jnp.float32)

    @pl.when(s == 0)
    def _():
        for k in range(15):
            pltpu.sync_copy(zb.at[pl.ds(0, 632)], acc.at[pl.ds(k * 632, 632)])
        pltpu.sync_copy(zb.at[pl.ds(0, 520)], acc.at[pl.ds(15 * 632, 520)])

    plsc.subcore_barrier()

    def chunk(i, carry):
        pltpu.sync_copy(dst_hbm.at[pl.ds(base + i * CH, CH)], didx)
        pltpu.sync_copy(ones_v, acc.at[didx], add=True)
        return carry

    lax.fori_loop(0, NFULL, chunk, 0)
    offt = base + NFULL * CH
    pltpu.sync_copy(dst_hbm.at[pl.ds(offt, TAIL)], didx_t)
    pltpu.sync_copy(ones_v.at[pl.ds(0, TAIL)], acc.at[didx_t], add=True)

    plsc.subcore_barrier()

    @pl.when(s == 0)
    def _():
        pltpu.sync_copy(acc, out_hbm.at[c])


@functools.partial(
    pl.kernel,
    out_type=jax.ShapeDtypeStruct((NC, N, H), jnp.float32),
    mesh=_mesh,
    scratch_types=[
        pltpu.VMEM((CH,), jnp.int32),
        pltpu.VMEM((CH,), jnp.int32),
        pltpu.VMEM((TAIL,), jnp.int32),
        pltpu.VMEM((TAIL,), jnp.int32),
        pltpu.VMEM((CH, H), jnp.float32),
        pltpu.VMEM((TAIL, H), jnp.float32),
        pltpu.VMEM_SHARED((N, H), jnp.float32),
        pltpu.SemaphoreType.DMA,
    ],
)
def _agg(hd_hbm, src_hbm, dst_hbm, out_hbm, sidx, didx, sidx_t, didx_t,
         rows, rows_t, acc, sem):
    c = lax.axis_index("c")
    s = lax.axis_index("s")
    w = c * NS + s
    base = w * EPT

    def zrow(r, carry):
        for jf in range(H // L):
            rows[r, pl.ds(jf * L, L)] = jnp.zeros((L,), jnp.float32)
        return carry

    lax.fori_loop(0, CH, zrow, 0)
    for k in range(4):
        pltpu.sync_copy(rows, acc.at[pl.ds(s * RPT + k * CH, CH)])
    pltpu.sync_copy(rows.at[pl.ds(0, RPT - 4 * CH)],
                    acc.at[pl.ds(s * RPT + 4 * CH, RPT - 4 * CH)])

    plsc.subcore_barrier()

    def chunk(i, carry):
        off = base + i * CH
        pltpu.sync_copy(src_hbm.at[pl.ds(off, CH)], sidx)
        pltpu.sync_copy(dst_hbm.at[pl.ds(off, CH)], didx)
        pltpu.async_copy(hd_hbm.at[sidx], rows, sem).wait()
        pltpu.sync_copy(rows, acc.at[didx], add=True)
        return carry

    lax.fori_loop(0, NFULL, chunk, 0)
    offt = base + NFULL * CH
    pltpu.sync_copy(src_hbm.at[pl.ds(offt, TAIL)], sidx_t)
    pltpu.sync_copy(dst_hbm.at[pl.ds(offt, TAIL)], didx_t)
    pltpu.async_copy(hd_hbm.at[sidx_t], rows_t, sem).wait()
    pltpu.sync_copy(rows_t, acc.at[didx_t], add=True)

    plsc.subcore_barrier()
    pltpu.sync_copy(acc.at[pl.ds(s * RPT, RPT)],
                    out_hbm.at[c, pl.ds(s * RPT, RPT)])


@functools.partial(
    pl.kernel,
    out_type=jax.ShapeDtypeStruct((NW, G, H), jnp.float32),
    mesh=_mesh,
    scratch_types=[
        pltpu.VMEM((NPT, H), jnp.float32),
        pltpu.VMEM((NPT,), jnp.int32),
        pltpu.VMEM((G, H), jnp.float32),
    ],
)
def _maxpool(h_hbm, batch_hbm, out_hbm, rows, bids, acc):
    c = lax.axis_index("c")
    s = lax.axis_index("s")
    w = c * NS + s
    base = w * NPT

    neg = jnp.full((L,), -jnp.inf, jnp.float32)

    def zrow(r, carry):
        for jf in range(H // L):
            acc[r, pl.ds(jf * L, L)] = neg
        return carry

    lax.fori_loop(0, G, zrow, 0)

    pltpu.sync_copy(h_hbm.at[pl.ds(base, NPT)], rows)
    pltpu.sync_copy(batch_hbm.at[pl.ds(base, NPT)], bids)

    cols = [lax.iota(jnp.int32, (L,)) + jf * L for jf in range(H // L)]

    def node(i, carry):
        isplat = jnp.zeros((L,), jnp.int32) + i
        b = plsc.load_gather(bids, [isplat])
        for jf in range(H // L):
            rv = plsc.load_gather(rows, [isplat, cols[jf]])
            cur = plsc.load_gather(acc, [b, cols[jf]])
            plsc.store_scatter(acc, [b, cols[jf]], jnp.maximum(cur, rv))
        return carry

    lax.fori_loop(0, NPT, node, 0)

    @pl.when(w == 0)
    def _():
        pltpu.sync_copy(h_hbm.at[pl.ds(NW * NPT, REST)], rows.at[pl.ds(0, REST)])
        pltpu.sync_copy(batch_hbm.at[pl.ds(NW * NPT, REST)],
                        bids.at[pl.ds(0, REST)])
        lax.fori_loop(0, REST, node, 0)

    pltpu.sync_copy(acc, out_hbm.at[w])


def _tc_pre_body(degp, x, w1, hd, dinv):
    d = degp[...]
    deg = d[0] + d[1] + 1.0
    dv = lax.rsqrt(deg)
    p = jnp.dot(x[...], w1[...], preferred_element_type=jnp.float32)
    hd[...] = p * dv
    dinv[...] = dv


def _tc_mid_body(sp, hd, dinv, b, gg, be, w, out):
    spv = sp[...]
    dv = dinv[...]
    agg = (spv[0] + spv[1] + hd[...]) * dv + b[...][None, :]
    mu = jnp.mean(agg, axis=0, keepdims=True)
    xc = agg - mu
    var = jnp.mean(xc * xc, axis=0, keepdims=True)
    h = jnp.maximum(xc * lax.rsqrt(var + 1e-5) * gg[...][None, :]
                    + be[...][None, :], 0.0)
    out[...] = jnp.dot(h, w[...], preferred_element_type=jnp.float32) * dv


def _tc_post_body(sp, hd, dinv, b, gg, be, out):
    spv = sp[...]
    agg = (spv[0] + spv[1] + hd[...]) * dinv[...] + b[...][None, :]
    mu = jnp.mean(agg, axis=0, keepdims=True)
    xc = agg - mu
    var = jnp.mean(xc * xc, axis=0, keepdims=True)
    out[...] = jnp.maximum(xc * lax.rsqrt(var + 1e-5) * gg[...][None, :]
                           + be[...][None, :], 0.0)


def _tc_out_body(h3, batch, mp, wout, bout, out):
    bt = batch[...]
    gids = lax.broadcasted_iota(jnp.int32, (G, N), 0)
    mask = (bt[None, :] == gids).astype(jnp.float32)
    cnt = jnp.sum(mask, axis=1, keepdims=True)
    mean = jnp.dot(mask, h3[...], preferred_element_type=jnp.float32) \
        / jnp.maximum(cnt, 1.0)
    mx = jnp.max(mp[...], axis=0)
    mx = jnp.where(cnt > 0, mx, 0.0)
    wo = wout[...]
    out[...] = (jnp.dot(mean, wo[:H], preferred_element_type=jnp.float32)
                + jnp.dot(mx, wo[H:], preferred_element_type=jnp.float32)
                + bout[...][None, :])


def _tc_call(body, out_shape, *args):
    return pl.pallas_call(body, out_shape=out_shape)(*args)


def kernel(x, edge_index, batch, W1, b1, g1, be1, W2, b2, g2, be2,
           W3, b3, g3, be3, Wout, bout):
    src = edge_index[0]
    dst = edge_index[1]

    degp = _deg(dst).reshape(NC, N, 1)
    hd1, dinv = _tc_call(
        _tc_pre_body,
        (jax.ShapeDtypeStruct((N, H), jnp.float32),
         jax.ShapeDtypeStruct((N, 1), jnp.float32)),
        degp, x, W1)

    s1 = _agg(hd1, src, dst)
    hd2 = _tc_call(_tc_mid_body, jax.ShapeDtypeStruct((N, H), jnp.float32),
                   s1, hd1, dinv, b1, g1, be1, W2)
    s2 = _agg(hd2, src, dst)
    hd3 = _tc_call(_tc_mid_body, jax.ShapeDtypeStruct((N, H), jnp.float32),
                   s2, hd2, dinv, b2, g2, be2, W3)
    s3 = _agg(hd3, src, dst)
    h3 = _tc_call(_tc_post_body, jax.ShapeDtypeStruct((N, H), jnp.float32),
                  s3, hd3, dinv, b3, g3, be3)

    mp = _maxpool(h3, batch)
    return _tc_call(_tc_out_body, jax.ShapeDtypeStruct((G, 1), jnp.float32),
                    h3, batch, mp, Wout, bout)


# speedup vs baseline: 14.0665x; 14.0665x over previous
import functools

import jax
import jax.numpy as jnp
from jax import lax
from jax.experimental import pallas as pl
from jax.experimental.pallas import tpu as pltpu
from jax.experimental.pallas import tpu_sc as plsc

N = 10000
E = 320000
F = 128
H = 64
G = 128

NC = 2
NS = 16
L = 16
NW = NC * NS

EPT = E // NW
CH = 128
NFULL = EPT // CH
TAIL = EPT - NFULL * CH
HP = 128
RPT = N // NS
STRIPE = 632
LAST_STRIPE = N - (NS - 1) * STRIPE

NPT = 312
REST = N - NW * NPT

_mesh = plsc.VectorSubcoreMesh(core_axis_name="c", subcore_axis_name="s",
                               num_cores=NC, num_subcores=NS)


@functools.partial(
    pl.kernel,
    out_type=jax.ShapeDtypeStruct((NC, 1, N), jnp.float32),
    mesh=_mesh,
    scratch_types=[
        pltpu.VMEM((CH,), jnp.int32),
        pltpu.VMEM((TAIL,), jnp.int32),
        pltpu.VMEM((CH,), jnp.float32),
        pltpu.VMEM((640,), jnp.float32),
        pltpu.VMEM_SHARED((N,), jnp.float32),
    ],
)
def _deg(dst_hbm, out_hbm, didx, didx_t, ones_v, zb, acc):
    c = lax.axis_index("c")
    s = lax.axis_index("s")
    w = c * NS + s
    base = w * EPT

    def z16(i, carry):
        zb[pl.ds(i * L, L)] = jnp.zeros((L,), jnp.float32)
        return carry

    lax.fori_loop(0, 640 // L, z16, 0)
    for j in range(CH // L):
        ones_v[pl.ds(j * L, L)] = jnp.ones((L,), jnp.float32)

    @pl.when(s == 0)
    def _():
        for k in range(15):
            pltpu.sync_copy(zb.at[pl.ds(0, 632)], acc.at[pl.ds(k * 632, 632)])
        pltpu.sync_copy(zb.at[pl.ds(0, 520)], acc.at[pl.ds(15 * 632, 520)])

    plsc.subcore_barrier()

    def chunk(i, carry):
        pltpu.sync_copy(dst_hbm.at[pl.ds(base + i * CH, CH)], didx)
        pltpu.sync_copy(ones_v, acc.at[didx], add=True)
        return carry

    lax.fori_loop(0, NFULL, chunk, 0)
    offt = base + NFULL * CH
    pltpu.sync_copy(dst_hbm.at[pl.ds(offt, TAIL)], didx_t)
    pltpu.sync_copy(ones_v.at[pl.ds(0, TAIL)], acc.at[didx_t], add=True)

    plsc.subcore_barrier()

    @pl.when(s == 0)
    def _():
        pltpu.sync_copy(acc, out_hbm.at[c, 0])


@functools.partial(
    pl.kernel,
    out_type=jax.ShapeDtypeStruct((NC, N, HP), jnp.float32),
    mesh=_mesh,
    scratch_types=[
        pltpu.VMEM((CH,), jnp.int32),
        pltpu.VMEM((CH,), jnp.int32),
        pltpu.VMEM((TAIL,), jnp.int32),
        pltpu.VMEM((TAIL,), jnp.int32),
        pltpu.VMEM((CH, HP), jnp.float32),
        pltpu.VMEM((TAIL, HP), jnp.float32),
        pltpu.VMEM_SHARED((N, HP), jnp.float32),
        pltpu.SemaphoreType.DMA,
    ],
)
def _agg(hd_hbm, src_hbm, dst_hbm, out_hbm, sidx, didx, sidx_t, didx_t,
         rows, rows_t, acc, sem):
    c = lax.axis_index("c")
    s = lax.axis_index("s")
    w = c * NS + s
    base = w * EPT

    def zrow(r, carry):
        for jf in range(HP // L):
            rows[r, pl.ds(jf * L, L)] = jnp.zeros((L,), jnp.float32)
        return carry

    lax.fori_loop(0, CH, zrow, 0)

    start = s * STRIPE

    def zero_stripe(nrows):
        for k in range(nrows // CH):
            pltpu.sync_copy(rows, acc.at[pl.ds(start + k * CH, CH)])
        rem = nrows % CH
        pltpu.sync_copy(rows.at[pl.ds(0, rem)],
                        acc.at[pl.ds(start + (nrows // CH) * CH, rem)])

    @pl.when(s < NS - 1)
    def _():
        zero_stripe(STRIPE)

    @pl.when(s == NS - 1)
    def _():
        zero_stripe(LAST_STRIPE)

    plsc.subcore_barrier()

    def chunk(i, carry):
        off = base + i * CH
        pltpu.sync_copy(src_hbm.at[pl.ds(off, CH)], sidx)
        pltpu.sync_copy(dst_hbm.at[pl.ds(off, CH)], didx)
        pltpu.async_copy(hd_hbm.at[sidx], rows, sem).wait()
        pltpu.sync_copy(rows, acc.at[didx], add=True)
        return carry

    lax.fori_loop(0, NFULL, chunk, 0)
    offt = base + NFULL * CH
    pltpu.sync_copy(src_hbm.at[pl.ds(offt, TAIL)], sidx_t)
    pltpu.sync_copy(dst_hbm.at[pl.ds(offt, TAIL)], didx_t)
    pltpu.async_copy(hd_hbm.at[sidx_t], rows_t, sem).wait()
    pltpu.sync_copy(rows_t, acc.at[didx_t], add=True)

    plsc.subcore_barrier()

    @pl.when(s < NS - 1)
    def _():
        pltpu.sync_copy(acc.at[pl.ds(start, STRIPE)],
                        out_hbm.at[c, pl.ds(start, STRIPE)])

    @pl.when(s == NS - 1)
    def _():
        pltpu.sync_copy(acc.at[pl.ds(start, LAST_STRIPE)],
                        out_hbm.at[c, pl.ds(start, LAST_STRIPE)])


@functools.partial(
    pl.kernel,
    out_type=jax.ShapeDtypeStruct((NW, G, H), jnp.float32),
    mesh=_mesh,
    scratch_types=[
        pltpu.VMEM((NPT, H), jnp.float32),
        pltpu.VMEM((NPT + L,), jnp.int32),
        pltpu.VMEM((G, H), jnp.float32),
    ],
)
def _maxpool(h_hbm, batch_hbm, out_hbm, rows, bids, acc):
    c = lax.axis_index("c")
    s = lax.axis_index("s")
    w = c * NS + s
    base = w * NPT

    neg = jnp.full((L,), -jnp.inf, jnp.float32)

    def zrow(r, carry):
        for jf in range(H // L):
            acc[r, pl.ds(jf * L, L)] = neg
        return carry

    lax.fori_loop(0, G, zrow, 0)

    pltpu.sync_copy(h_hbm.at[pl.ds(base, NPT)], rows)
    pltpu.sync_copy(batch_hbm.at[pl.ds(base, NPT)], bids.at[pl.ds(0, NPT)])

    def node(i, carry):
        b = bids[pl.ds(i, L)][0]
        for jf in range(H // L):
            rv = rows[i, pl.ds(jf * L, L)]
            cur = acc[b, pl.ds(jf * L, L)]
            acc[b, pl.ds(jf * L, L)] = jnp.maximum(cur, rv)
        return carry

    lax.fori_loop(0, NPT, node, 0)

    @pl.when(w == 0)
    def _():
        pltpu.sync_copy(h_hbm.at[pl.ds(NW * NPT, REST)], rows.at[pl.ds(0, REST)])
        pltpu.sync_copy(batch_hbm.at[pl.ds(NW * NPT, REST)],
                        bids.at[pl.ds(0, REST)])
        lax.fori_loop(0, REST, node, 0)

    pltpu.sync_copy(acc, out_hbm.at[w])


def _tc_pre_body(degp, x, w1, hd, dinv):
    d = degp[...]
    deg = d[0] + d[1] + 1.0
    dv = jnp.reshape(lax.rsqrt(deg), (N, 1))
    p = jnp.dot(x[...], w1[...], preferred_element_type=jnp.float32) * dv
    hd[...] = jnp.concatenate(
        [p, jnp.zeros((N, HP - H), jnp.float32)], axis=1)
    dinv[...] = dv


def _tc_mid_body(sp, hd, dinv, b, gg, be, w, out):
    spv = sp[...]
    dv = dinv[...]
    agg = ((spv[0, :, :H] + spv[1, :, :H] + hd[..., :H]) * dv
           + b[...][None, :])
    mu = jnp.mean(agg, axis=0, keepdims=True)
    xc = agg - mu
    var = jnp.mean(xc * xc, axis=0, keepdims=True)
    h = jnp.maximum(xc * lax.rsqrt(var + 1e-5) * gg[...][None, :]
                    + be[...][None, :], 0.0)
    p = jnp.dot(h, w[...], preferred_element_type=jnp.float32) * dv
    out[...] = jnp.concatenate(
        [p, jnp.zeros((N, HP - H), jnp.float32)], axis=1)


def _tc_post_body(sp, hd, dinv, b, gg, be, out):
    spv = sp[...]
    agg = ((spv[0, :, :H] + spv[1, :, :H] + hd[..., :H]) * dinv[...]
           + b[...][None, :])
    mu = jnp.mean(agg, axis=0, keepdims=True)
    xc = agg - mu
    var = jnp.mean(xc * xc, axis=0, keepdims=True)
    out[...] = jnp.maximum(xc * lax.rsqrt(var + 1e-5) * gg[...][None, :]
                           + be[...][None, :], 0.0)


def _tc_out_body(h3, batch, mp, wout, bout, out):
    bt = batch[...]
    gids = lax.broadcasted_iota(jnp.int32, (G, N), 0)
    mask = (bt[None, :] == gids).astype(jnp.float32)
    cnt = jnp.sum(mask, axis=1, keepdims=True)
    mean = jnp.dot(mask, h3[...], preferred_element_type=jnp.float32) \
        / jnp.maximum(cnt, 1.0)
    mx = jnp.max(mp[...], axis=0)
    mx = jnp.where(cnt > 0, mx, 0.0)
    wo = wout[...]
    out[...] = (jnp.dot(mean, wo[:H], preferred_element_type=jnp.float32)
                + jnp.dot(mx, wo[H:], preferred_element_type=jnp.float32)
                + bout[...][None, :])


def _tc_call(body, out_shape, *args):
    return pl.pallas_call(body, out_shape=out_shape)(*args)


def kernel(x, edge_index, batch, W1, b1, g1, be1, W2, b2, g2, be2,
           W3, b3, g3, be3, Wout, bout):
    src = edge_index[0]
    dst = edge_index[1]

    degp = _deg(dst)
    hd1, dinv = _tc_call(
        _tc_pre_body,
        (jax.ShapeDtypeStruct((N, HP), jnp.float32),
         jax.ShapeDtypeStruct((N, 1), jnp.float32)),
        degp, x, W1)

    s1 = _agg(hd1, src, dst)
    hd2 = _tc_call(_tc_mid_body, jax.ShapeDtypeStruct((N, HP), jnp.float32),
                   s1, hd1, dinv, b1, g1, be1, W2)
    s2 = _agg(hd2, src, dst)
    hd3 = _tc_call(_tc_mid_body, jax.ShapeDtypeStruct((N, HP), jnp.float32),
                   s2, hd2, dinv, b2, g2, be2, W3)
    s3 = _agg(hd3, src, dst)
    h3 = _tc_call(_tc_post_body, jax.ShapeDtypeStruct((N, H), jnp.float32),
                  s3, hd3, dinv, b3, g3, be3)

    mp = _maxpool(h3, batch)
    return _tc_call(_tc_out_body, jax.ShapeDtypeStruct((G, 1), jnp.float32),
                    h3, batch, mp, Wout, bout)


# speedup vs baseline: 29.0695x; 2.0666x over previous
import functools

import jax
import jax.numpy as jnp
from jax import lax
from jax.experimental import pallas as pl
from jax.experimental.pallas import tpu as pltpu
from jax.experimental.pallas import tpu_sc as plsc

N = 10000
E = 320000
F = 128
H = 64
G = 128

NC = 2
NS = 16
L = 16
NW = NC * NS

CH = 128
NCHUNK = E // CH
NF2 = NCHUNK // NW
XTRA = NCHUNK - NW * NF2
PAIRS = NF2 // 2

STRIPE = 632
LAST_STRIPE = N - (NS - 1) * STRIPE

NPT = 312
REST = N - NW * NPT

_mesh = plsc.VectorSubcoreMesh(core_axis_name="c", subcore_axis_name="s",
                               num_cores=NC, num_subcores=NS)
_sc_params = pltpu.CompilerParams(use_tc_tiling_on_sc=False)


@functools.partial(
    pl.kernel,
    out_type=jax.ShapeDtypeStruct((NC, 1, N), jnp.float32),
    mesh=_mesh,
    compiler_params=_sc_params,
    scratch_types=[
        pltpu.VMEM((NF2, 1, CH), jnp.int32),
        pltpu.VMEM((1, CH), jnp.int32),
        pltpu.VMEM((CH,), jnp.float32),
        pltpu.VMEM((640,), jnp.float32),
        pltpu.VMEM_SHARED((N,), jnp.float32),
        pltpu.SemaphoreType.DMA,
    ],
)
def _deg(dst_hbm, out_hbm, didx_all, didx_x, ones_v, zb, acc, sd):
    c = lax.axis_index("c")
    s = lax.axis_index("s")
    w = c * NS + s
    cbase = w * NF2

    def z16(i, carry):
        zb[pl.ds(i * L, L)] = jnp.zeros((L,), jnp.float32)
        return carry

    lax.fori_loop(0, 640 // L, z16, 0)
    for j in range(CH // L):
        ones_v[pl.ds(j * L, L)] = jnp.ones((L,), jnp.float32)

    @pl.when(s == 0)
    def _():
        for k in range(15):
            pltpu.sync_copy(zb.at[pl.ds(0, 632)], acc.at[pl.ds(k * 632, 632)])
        pltpu.sync_copy(zb.at[pl.ds(0, 520)], acc.at[pl.ds(15 * 632, 520)])

    pltpu.sync_copy(dst_hbm.at[pl.ds(cbase, NF2)], didx_all)

    plsc.subcore_barrier()

    def fire(j, carry):
        pltpu.async_copy(ones_v, acc.at[didx_all.at[j, 0]], sd, add=True)
        return carry

    lax.fori_loop(0, NF2, fire, 0)

    @pl.when(w < XTRA)
    def _():
        pltpu.sync_copy(dst_hbm.at[NW * NF2 + w], didx_x)
        pltpu.async_copy(ones_v, acc.at[didx_x.at[0]], sd, add=True)

    def drain(j, carry):
        pltpu.make_async_copy(ones_v, acc.at[didx_all.at[j, 0]], sd).wait()
        return carry

    lax.fori_loop(0, NF2, drain, 0)

    @pl.when(w < XTRA)
    def _():
        pltpu.make_async_copy(ones_v, acc.at[didx_x.at[0]], sd).wait()

    plsc.subcore_barrier()

    @pl.when(s == 0)
    def _():
        pltpu.sync_copy(acc, out_hbm.at[c, 0])


@functools.partial(
    pl.kernel,
    out_type=jax.ShapeDtypeStruct((NC, N, H), jnp.float32),
    mesh=_mesh,
    compiler_params=_sc_params,
    scratch_types=[
        pltpu.VMEM((NF2, 1, CH), jnp.int32),
        pltpu.VMEM((NF2, 1, CH), jnp.int32),
        pltpu.VMEM((1, CH), jnp.int32),
        pltpu.VMEM((1, CH), jnp.int32),
        pltpu.VMEM((CH, H), jnp.float32),
        pltpu.VMEM((CH, H), jnp.float32),
        pltpu.VMEM_SHARED((N, H), jnp.float32),
        pltpu.SemaphoreType.DMA,
        pltpu.SemaphoreType.DMA,
        pltpu.SemaphoreType.DMA,
    ],
)
def _agg(hd_hbm, src_hbm, dst_hbm, out_hbm, sidx_all, didx_all, sidx_x,
         didx_x, rows0, rows1, acc, gs, ss0, ss1):
    c = lax.axis_index("c")
    s = lax.axis_index("s")
    w = c * NS + s
    cbase = w * NF2

    def zrow(r, carry):
        for jf in range(H // L):
            rows0[r, pl.ds(jf * L, L)] = jnp.zeros((L,), jnp.float32)
        return carry

    lax.fori_loop(0, CH, zrow, 0)

    start = s * STRIPE

    def zero_stripe(nrows):
        for k in range(nrows // CH):
            pltpu.sync_copy(rows0, acc.at[pl.ds(start + k * CH, CH)])
        rem = nrows % CH
        pltpu.sync_copy(rows0.at[pl.ds(0, rem)],
                        acc.at[pl.ds(start + (nrows // CH) * CH, rem)])

    @pl.when(s < NS - 1)
    def _():
        zero_stripe(STRIPE)

    @pl.when(s == NS - 1)
    def _():
        zero_stripe(LAST_STRIPE)

    pltpu.sync_copy(src_hbm.at[pl.ds(cbase, NF2)], sidx_all)
    pltpu.sync_copy(dst_hbm.at[pl.ds(cbase, NF2)], didx_all)

    plsc.subcore_barrier()

    bufs = (rows0, rows1)
    ssems = (ss0, ss1)
    pltpu.async_copy(hd_hbm.at[sidx_all.at[0, 0]], rows0, gs)

    def pair(g, carry):
        for b in (0, 1):
            j = 2 * g + b
            rb, rn = bufs[b], bufs[1 - b]
            pltpu.make_async_copy(hd_hbm.at[sidx_all.at[j, 0]], rb, gs).wait()

            @pl.when(j >= 1)
            def _():
                pltpu.make_async_copy(rn, acc.at[didx_all.at[j - 1, 0]],
                                      ssems[1 - b]).wait()

            if b == 0:
                pltpu.async_copy(hd_hbm.at[sidx_all.at[j + 1, 0]], rn, gs)
            else:
                @pl.when(g < PAIRS - 1)
                def _():
                    pltpu.async_copy(hd_hbm.at[sidx_all.at[j + 1, 0]], rn, gs)

            pltpu.async_copy(rb, acc.at[didx_all.at[j, 0]], ssems[b], add=True)
        return carry

    lax.fori_loop(0, PAIRS, pair, 0)
    pltpu.make_async_copy(rows1, acc.at[didx_all.at[NF2 - 1, 0]], ss1).wait()

    @pl.when(w < XTRA)
    def _():
        xid = NW * NF2 + w
        pltpu.sync_copy(src_hbm.at[xid], sidx_x)
        pltpu.sync_copy(dst_hbm.at[xid], didx_x)
        pltpu.async_copy(hd_hbm.at[sidx_x.at[0]], rows0, gs).wait()
        pltpu.sync_copy(rows0, acc.at[didx_x.at[0]], add=True)

    plsc.subcore_barrier()

    @pl.when(s < NS - 1)
    def _():
        pltpu.sync_copy(acc.at[pl.ds(start, STRIPE)],
                        out_hbm.at[c, pl.ds(start, STRIPE)])

    @pl.when(s == NS - 1)
    def _():
        pltpu.sync_copy(acc.at[pl.ds(start, LAST_STRIPE)],
                        out_hbm.at[c, pl.ds(start, LAST_STRIPE)])


@functools.partial(
    pl.kernel,
    out_type=jax.ShapeDtypeStruct((NW, G, H), jnp.float32),
    mesh=_mesh,
    compiler_params=_sc_params,
    scratch_types=[
        pltpu.VMEM((NPT, H), jnp.float32),
        pltpu.VMEM((NPT + L,), jnp.int32),
        pltpu.VMEM((G, H), jnp.float32),
    ],
)
def _maxpool(h_hbm, batch_hbm, out_hbm, rows, bids, acc):
    c = lax.axis_index("c")
    s = lax.axis_index("s")
    w = c * NS + s
    base = w * NPT

    neg = jnp.full((L,), -jnp.inf, jnp.float32)

    def zrow(r, carry):
        for jf in range(H // L):
            acc[r, pl.ds(jf * L, L)] = neg
        return carry

    lax.fori_loop(0, G, zrow, 0)

    pltpu.sync_copy(h_hbm.at[pl.ds(base, NPT)], rows)
    pltpu.sync_copy(batch_hbm.at[pl.ds(base, NPT)], bids.at[pl.ds(0, NPT)])

    def node(i, carry):
        b = bids[pl.ds(i, L)][0]
        for jf in range(H // L):
            rv = rows[i, pl.ds(jf * L, L)]
            cur = acc[b, pl.ds(jf * L, L)]
            acc[b, pl.ds(jf * L, L)] = jnp.maximum(cur, rv)
        return carry

    lax.fori_loop(0, NPT, node, 0)

    @pl.when(w == 0)
    def _():
        pltpu.sync_copy(h_hbm.at[pl.ds(NW * NPT, REST)], rows.at[pl.ds(0, REST)])
        pltpu.sync_copy(batch_hbm.at[pl.ds(NW * NPT, REST)],
                        bids.at[pl.ds(0, REST)])
        lax.fori_loop(0, REST, node, 0)

    pltpu.sync_copy(acc, out_hbm.at[w])


def _tc_pre_body(degp, x, w1, hd, dinv):
    d = degp[...]
    deg = d[0] + d[1] + 1.0
    dv = jnp.reshape(lax.rsqrt(deg), (N, 1))
    p = jnp.dot(x[...], w1[...], preferred_element_type=jnp.float32)
    hd[...] = p * dv
    dinv[...] = dv


def _tc_mid_body(sp, hd, dinv, b, gg, be, w, out):
    spv = sp[...]
    dv = dinv[...]
    agg = (spv[0] + spv[1] + hd[...]) * dv + b[...][None, :]
    mu = jnp.mean(agg, axis=0, keepdims=True)
    xc = agg - mu
    var = jnp.mean(xc * xc, axis=0, keepdims=True)
    h = jnp.maximum(xc * lax.rsqrt(var + 1e-5) * gg[...][None, :]
                    + be[...][None, :], 0.0)
    out[...] = jnp.dot(h, w[...], preferred_element_type=jnp.float32) * dv


def _tc_post_body(sp, hd, dinv, b, gg, be, out):
    spv = sp[...]
    agg = (spv[0] + spv[1] + hd[...]) * dinv[...] + b[...][None, :]
    mu = jnp.mean(agg, axis=0, keepdims=True)
    xc = agg - mu
    var = jnp.mean(xc * xc, axis=0, keepdims=True)
    out[...] = jnp.maximum(xc * lax.rsqrt(var + 1e-5) * gg[...][None, :]
                           + be[...][None, :], 0.0)


def _tc_out_body(h3, batch, mp, wout, bout, out):
    bt = batch[...]
    gids = lax.broadcasted_iota(jnp.int32, (G, N), 0)
    mask = (bt[None, :] == gids).astype(jnp.float32)
    cnt = jnp.sum(mask, axis=1, keepdims=True)
    mean = jnp.dot(mask, h3[...], preferred_element_type=jnp.float32) \
        / jnp.maximum(cnt, 1.0)
    mx = jnp.max(mp[...], axis=0)
    mx = jnp.where(cnt > 0, mx, 0.0)
    wo = wout[...]
    out[...] = (jnp.dot(mean, wo[:H], preferred_element_type=jnp.float32)
                + jnp.dot(mx, wo[H:], preferred_element_type=jnp.float32)
                + bout[...][None, :])


def _tc_call(body, out_shape, *args):
    return pl.pallas_call(body, out_shape=out_shape)(*args)


def kernel(x, edge_index, batch, W1, b1, g1, be1, W2, b2, g2, be2,
           W3, b3, g3, be3, Wout, bout):
    src3 = edge_index[0].reshape(NCHUNK, 1, CH)
    dst3 = edge_index[1].reshape(NCHUNK, 1, CH)

    degp = _deg(dst3)
    hd1, dinv = _tc_call(
        _tc_pre_body,
        (jax.ShapeDtypeStruct((N, H), jnp.float32),
         jax.ShapeDtypeStruct((N, 1), jnp.float32)),
        degp, x, W1)

    s1 = _agg(hd1, src3, dst3)
    hd2 = _tc_call(_tc_mid_body, jax.ShapeDtypeStruct((N, H), jnp.float32),
                   s1, hd1, dinv, b1, g1, be1, W2)
    s2 = _agg(hd2, src3, dst3)
    hd3 = _tc_call(_tc_mid_body, jax.ShapeDtypeStruct((N, H), jnp.float32),
                   s2, hd2, dinv, b2, g2, be2, W3)
    s3 = _agg(hd3, src3, dst3)
    h3 = _tc_call(_tc_post_body, jax.ShapeDtypeStruct((N, H), jnp.float32),
                  s3, hd3, dinv, b3, g3, be3)

    mp = _maxpool(h3, batch)
    return _tc_call(_tc_out_body, jax.ShapeDtypeStruct((G, 1), jnp.float32),
                    h3, batch, mp, Wout, bout)


# speedup vs baseline: 31.1209x; 1.0706x over previous
import functools

import jax
import jax.numpy as jnp
from jax import lax
from jax.experimental import pallas as pl
from jax.experimental.pallas import tpu as pltpu
from jax.experimental.pallas import tpu_sc as plsc

N = 10000
E = 320000
F = 128
H = 64
G = 128

NC = 2
NS = 16
L = 16
NW = NC * NS

CH = 128
NCHUNK = E // CH
NF2 = NCHUNK // NW
XTRA = NCHUNK - NW * NF2
PAIRS = NF2 // 2

STRIPE = 632
LAST_STRIPE = N - (NS - 1) * STRIPE

NPT = 312
REST = N - NW * NPT

_mesh = plsc.VectorSubcoreMesh(core_axis_name="c", subcore_axis_name="s",
                               num_cores=NC, num_subcores=NS)
_sc_params = pltpu.CompilerParams(use_tc_tiling_on_sc=False)


@functools.partial(
    pl.kernel,
    out_type=jax.ShapeDtypeStruct((NC, 1, N), jnp.float32),
    mesh=_mesh,
    compiler_params=_sc_params,
    scratch_types=[
        pltpu.VMEM((NF2, 1, CH), jnp.int32),
        pltpu.VMEM((1, CH), jnp.int32),
        pltpu.VMEM((CH,), jnp.float32),
        pltpu.VMEM((640,), jnp.float32),
        pltpu.VMEM_SHARED((N,), jnp.float32),
        pltpu.SemaphoreType.DMA,
    ],
)
def _deg(dst_hbm, out_hbm, didx_all, didx_x, ones_v, zb, acc, sd):
    c = lax.axis_index("c")
    s = lax.axis_index("s")
    w = c * NS + s
    cbase = w * NF2

    def z16(i, carry):
        zb[pl.ds(i * L, L)] = jnp.zeros((L,), jnp.float32)
        return carry

    lax.fori_loop(0, 640 // L, z16, 0)
    for j in range(CH // L):
        ones_v[pl.ds(j * L, L)] = jnp.ones((L,), jnp.float32)

    @pl.when(s == 0)
    def _():
        for k in range(15):
            pltpu.sync_copy(zb.at[pl.ds(0, 632)], acc.at[pl.ds(k * 632, 632)])
        pltpu.sync_copy(zb.at[pl.ds(0, 520)], acc.at[pl.ds(15 * 632, 520)])

    pltpu.sync_copy(dst_hbm.at[pl.ds(cbase, NF2)], didx_all)

    plsc.subcore_barrier()

    def fire(j, carry):
        pltpu.async_copy(ones_v, acc.at[didx_all.at[j, 0]], sd, add=True)
        return carry

    lax.fori_loop(0, NF2, fire, 0)

    @pl.when(w < XTRA)
    def _():
        pltpu.sync_copy(dst_hbm.at[NW * NF2 + w], didx_x)
        pltpu.async_copy(ones_v, acc.at[didx_x.at[0]], sd, add=True)

    def drain(j, carry):
        pltpu.make_async_copy(ones_v, acc.at[didx_all.at[j, 0]], sd).wait()
        return carry

    lax.fori_loop(0, NF2, drain, 0)

    @pl.when(w < XTRA)
    def _():
        pltpu.make_async_copy(ones_v, acc.at[didx_x.at[0]], sd).wait()

    plsc.subcore_barrier()

    @pl.when(s == 0)
    def _():
        pltpu.sync_copy(acc, out_hbm.at[c, 0])


@functools.partial(
    pl.kernel,
    out_type=jax.ShapeDtypeStruct((NC, N, H), jnp.float32),
    mesh=_mesh,
    compiler_params=_sc_params,
    scratch_types=[
        pltpu.VMEM((NF2, 1, CH), jnp.int32),
        pltpu.VMEM((NF2, 1, CH), jnp.int32),
        pltpu.VMEM((1, CH), jnp.int32),
        pltpu.VMEM((1, CH), jnp.int32),
        pltpu.VMEM((CH, H), jnp.float32),
        pltpu.VMEM((CH, H), jnp.float32),
        pltpu.VMEM_SHARED((N, H), jnp.float32),
        pltpu.VMEM_SHARED((N, H), jnp.float32),
        pltpu.SemaphoreType.DMA,
        pltpu.SemaphoreType.DMA,
        pltpu.SemaphoreType.DMA,
    ],
)
def _agg(hd_hbm, src_hbm, dst_hbm, out_hbm, sidx_all, didx_all, sidx_x,
         didx_x, rows0, rows1, acc, hd_spm, gs, ss0, ss1):
    c = lax.axis_index("c")
    s = lax.axis_index("s")
    w = c * NS + s
    cbase = w * NF2

    def zrow(r, carry):
        for jf in range(H // L):
            rows0[r, pl.ds(jf * L, L)] = jnp.zeros((L,), jnp.float32)
        return carry

    lax.fori_loop(0, CH, zrow, 0)

    start = s * STRIPE

    def zero_stripe(nrows):
        for k in range(nrows // CH):
            pltpu.sync_copy(rows0, acc.at[pl.ds(start + k * CH, CH)])
        rem = nrows % CH
        pltpu.sync_copy(rows0.at[pl.ds(0, rem)],
                        acc.at[pl.ds(start + (nrows // CH) * CH, rem)])

    @pl.when(s < NS - 1)
    def _():
        zero_stripe(STRIPE)
        pltpu.sync_copy(hd_hbm.at[pl.ds(start, STRIPE)],
                        hd_spm.at[pl.ds(start, STRIPE)])

    @pl.when(s == NS - 1)
    def _():
        zero_stripe(LAST_STRIPE)
        pltpu.sync_copy(hd_hbm.at[pl.ds(start, LAST_STRIPE)],
                        hd_spm.at[pl.ds(start, LAST_STRIPE)])

    pltpu.sync_copy(src_hbm.at[pl.ds(cbase, NF2)], sidx_all)
    pltpu.sync_copy(dst_hbm.at[pl.ds(cbase, NF2)], didx_all)

    plsc.subcore_barrier()

    bufs = (rows0, rows1)
    ssems = (ss0, ss1)
    pltpu.async_copy(hd_spm.at[sidx_all.at[0, 0]], rows0, gs)

    def pair(g, carry):
        for b in (0, 1):
            j = 2 * g + b
            rb, rn = bufs[b], bufs[1 - b]
            pltpu.make_async_copy(hd_spm.at[sidx_all.at[j, 0]], rb, gs).wait()

            @pl.when(j >= 1)
            def _():
                pltpu.make_async_copy(rn, acc.at[didx_all.at[j - 1, 0]],
                                      ssems[1 - b]).wait()

            if b == 0:
                pltpu.async_copy(hd_spm.at[sidx_all.at[j + 1, 0]], rn, gs)
            else:
                @pl.when(g < PAIRS - 1)
                def _():
                    pltpu.async_copy(hd_spm.at[sidx_all.at[j + 1, 0]], rn, gs)

            pltpu.async_copy(rb, acc.at[didx_all.at[j, 0]], ssems[b], add=True)
        return carry

    lax.fori_loop(0, PAIRS, pair, 0)
    pltpu.make_async_copy(rows1, acc.at[didx_all.at[NF2 - 1, 0]], ss1).wait()

    @pl.when(w < XTRA)
    def _():
        xid = NW * NF2 + w
        pltpu.sync_copy(src_hbm.at[xid], sidx_x)
        pltpu.sync_copy(dst_hbm.at[xid], didx_x)
        pltpu.async_copy(hd_spm.at[sidx_x.at[0]], rows0, gs).wait()
        pltpu.sync_copy(rows0, acc.at[didx_x.at[0]], add=True)

    plsc.subcore_barrier()

    @pl.when(s < NS - 1)
    def _():
        pltpu.sync_copy(acc.at[pl.ds(start, STRIPE)],
                        out_hbm.at[c, pl.ds(start, STRIPE)])

    @pl.when(s == NS - 1)
    def _():
        pltpu.sync_copy(acc.at[pl.ds(start, LAST_STRIPE)],
                        out_hbm.at[c, pl.ds(start, LAST_STRIPE)])


@functools.partial(
    pl.kernel,
    out_type=jax.ShapeDtypeStruct((NW, G, H), jnp.float32),
    mesh=_mesh,
    compiler_params=_sc_params,
    scratch_types=[
        pltpu.VMEM((NPT, H), jnp.float32),
        pltpu.VMEM((NPT + L,), jnp.int32),
        pltpu.VMEM((G, H), jnp.float32),
    ],
)
def _maxpool(h_hbm, batch_hbm, out_hbm, rows, bids, acc):
    c = lax.axis_index("c")
    s = lax.axis_index("s")
    w = c * NS + s
    base = w * NPT

    neg = jnp.full((L,), -jnp.inf, jnp.float32)

    def zrow(r, carry):
        for jf in range(H // L):
            acc[r, pl.ds(jf * L, L)] = neg
        return carry

    lax.fori_loop(0, G, zrow, 0)

    pltpu.sync_copy(h_hbm.at[pl.ds(base, NPT)], rows)
    pltpu.sync_copy(batch_hbm.at[pl.ds(base, NPT)], bids.at[pl.ds(0, NPT)])

    def node(i, carry):
        b = bids[pl.ds(i, L)][0]
        for jf in range(H // L):
            rv = rows[i, pl.ds(jf * L, L)]
            cur = acc[b, pl.ds(jf * L, L)]
            acc[b, pl.ds(jf * L, L)] = jnp.maximum(cur, rv)
        return carry

    lax.fori_loop(0, NPT, node, 0)

    @pl.when(w == 0)
    def _():
        pltpu.sync_copy(h_hbm.at[pl.ds(NW * NPT, REST)], rows.at[pl.ds(0, REST)])
        pltpu.sync_copy(batch_hbm.at[pl.ds(NW * NPT, REST)],
                        bids.at[pl.ds(0, REST)])
        lax.fori_loop(0, REST, node, 0)

    pltpu.sync_copy(acc, out_hbm.at[w])


def _tc_pre_body(degp, x, w1, hd, dinv):
    d = degp[...]
    deg = d[0] + d[1] + 1.0
    dv = jnp.reshape(lax.rsqrt(deg), (N, 1))
    p = jnp.dot(x[...], w1[...], preferred_element_type=jnp.float32)
    hd[...] = p * dv
    dinv[...] = dv


def _tc_mid_body(sp, hd, dinv, b, gg, be, w, out):
    spv = sp[...]
    dv = dinv[...]
    agg = (spv[0] + spv[1] + hd[...]) * dv + b[...][None, :]
    mu = jnp.mean(agg, axis=0, keepdims=True)
    xc = agg - mu
    var = jnp.mean(xc * xc, axis=0, keepdims=True)
    h = jnp.maximum(xc * lax.rsqrt(var + 1e-5) * gg[...][None, :]
                    + be[...][None, :], 0.0)
    out[...] = jnp.dot(h, w[...], preferred_element_type=jnp.float32) * dv


def _tc_post_body(sp, hd, dinv, b, gg, be, out):
    spv = sp[...]
    agg = (spv[0] + spv[1] + hd[...]) * dinv[...] + b[...][None, :]
    mu = jnp.mean(agg, axis=0, keepdims=True)
    xc = agg - mu
    var = jnp.mean(xc * xc, axis=0, keepdims=True)
    out[...] = jnp.maximum(xc * lax.rsqrt(var + 1e-5) * gg[...][None, :]
                           + be[...][None, :], 0.0)


def _tc_out_body(h3, batch, mp, wout, bout, out):
    bt = batch[...]
    gids = lax.broadcasted_iota(jnp.int32, (G, N), 0)
    mask = (bt[None, :] == gids).astype(jnp.float32)
    cnt = jnp.sum(mask, axis=1, keepdims=True)
    mean = jnp.dot(mask, h3[...], preferred_element_type=jnp.float32) \
        / jnp.maximum(cnt, 1.0)
    mx = jnp.max(mp[...], axis=0)
    mx = jnp.where(cnt > 0, mx, 0.0)
    wo = wout[...]
    out[...] = (jnp.dot(mean, wo[:H], preferred_element_type=jnp.float32)
                + jnp.dot(mx, wo[H:], preferred_element_type=jnp.float32)
                + bout[...][None, :])


def _tc_call(body, out_shape, *args):
    return pl.pallas_call(body, out_shape=out_shape)(*args)


def kernel(x, edge_index, batch, W1, b1, g1, be1, W2, b2, g2, be2,
           W3, b3, g3, be3, Wout, bout):
    src3 = edge_index[0].reshape(NCHUNK, 1, CH)
    dst3 = edge_index[1].reshape(NCHUNK, 1, CH)

    degp = _deg(dst3)
    hd1, dinv = _tc_call(
        _tc_pre_body,
        (jax.ShapeDtypeStruct((N, H), jnp.float32),
         jax.ShapeDtypeStruct((N, 1), jnp.float32)),
        degp, x, W1)

    s1 = _agg(hd1, src3, dst3)
    hd2 = _tc_call(_tc_mid_body, jax.ShapeDtypeStruct((N, H), jnp.float32),
                   s1, hd1, dinv, b1, g1, be1, W2)
    s2 = _agg(hd2, src3, dst3)
    hd3 = _tc_call(_tc_mid_body, jax.ShapeDtypeStruct((N, H), jnp.float32),
                   s2, hd2, dinv, b2, g2, be2, W3)
    s3 = _agg(hd3, src3, dst3)
    h3 = _tc_call(_tc_post_body, jax.ShapeDtypeStruct((N, H), jnp.float32),
                  s3, hd3, dinv, b3, g3, be3)

    mp = _maxpool(h3, batch)
    return _tc_call(_tc_out_body, jax.ShapeDtypeStruct((G, 1), jnp.float32),
                    h3, batch, mp, Wout, bout)
